# baseline (speedup 1.0000x reference)
import functools

import jax
import jax.numpy as jnp
from jax import lax
from jax.experimental import pallas as pl
from jax.experimental.pallas import tpu as pltpu
from jax.experimental.pallas import tpu_sc as plsc

_B = 16384
_C = 1000
_NC = 2
_NS = 16
_L = 16
_NW = _NC * _NS
_CPW = _B // _NW
_WR = 40
_NWIN = _C // _WR
_NBUF = 4

_mesh = plsc.VectorSubcoreMesh(core_axis_name="c", subcore_axis_name="s")


@functools.partial(
    pl.kernel,
    mesh=_mesh,
    out_type=jax.ShapeDtypeStruct((_NW, _L), jnp.float32),
    scratch_types=[
        pltpu.VMEM((_CPW,), jnp.int32),
        [pltpu.VMEM((_WR, _CPW), jnp.float32) for _ in range(_NBUF)],
        pltpu.VMEM((_L,), jnp.float32),
        [pltpu.SemaphoreType.DMA for _ in range(_NBUF)],
    ],
    compiler_params=pltpu.CompilerParams(use_tc_tiling_on_sc=True,
                                         needs_layout_passes=False),
)
def _stream_extract_t(pred_hbm, lab_hbm, out_hbm, lab_v, bufs, acc_v, sems):
    wid = lax.axis_index("s") * _NC + lax.axis_index("c")
    cbase = wid * _CPW
    pltpu.sync_copy(lab_hbm.at[pl.ds(cbase, _CPW)], lab_v)

    def issue(t):
        return pltpu.async_copy(
            pred_hbm.at[pl.ds(t * _WR, _WR), pl.ds(cbase, _CPW)],
            bufs[t % _NBUF], sems[t % _NBUF])

    iota = lax.iota(jnp.int32, _L)
    acc = jnp.zeros((_L,), jnp.float32)
    copies = [None] * _NBUF
    for t in range(min(_NBUF, _NWIN)):
        copies[t] = issue(t)
    for t in range(_NWIN):
        copies[t % _NBUF].wait()
        win = bufs[t % _NBUF]
        r0 = t * _WR
        for j in range(_CPW // _L):
            lab = lab_v[pl.ds(j * _L, _L)]
            u = lab - r0
            mask = (u >= 0) & (u < _WR)
            row_idx = jnp.minimum(jnp.maximum(u, 0), _WR - 1)
            col_idx = iota + j * _L
            val = plsc.load_gather(win, [row_idx, col_idx])
            acc = acc + jnp.where(mask, val, 0.0)
        nxt = t + _NBUF
        if nxt < _NWIN:
            copies[t % _NBUF] = issue(nxt)
    acc_v[...] = acc
    pltpu.sync_copy(acc_v, out_hbm.at[wid])


def kernel(predict, label):
    partial = _stream_extract_t(predict.T, label.astype(jnp.int32))
    return partial.sum() / predict.shape[0]


# speedup vs baseline: 2.1263x; 2.1263x over previous
import functools

import jax
import jax.numpy as jnp
from jax import lax
from jax.experimental import pallas as pl
from jax.experimental.pallas import tpu as pltpu
from jax.experimental.pallas import tpu_sc as plsc

_B = 16384
_C = 1000
_BLK = 2048
_NBLK = _B // _BLK


def _tc_body(lab_ref, pred_ref, out_ref, acc_ref):
    i = pl.program_id(0)
    lab = lab_ref[0, 0, :]
    rows = lax.broadcasted_iota(jnp.int32, (_C, _BLK), 0)
    sel = rows == lab[None, :]
    part = jnp.sum(jnp.where(sel, pred_ref[...], 0.0))

    @pl.when(i == 0)
    def _():
        acc_ref[0] = 0.0

    acc_ref[0] += part

    @pl.when(i == _NBLK - 1)
    def _():
        out_ref[0, 0] = acc_ref[0]


_tc_call = pl.pallas_call(
    _tc_body,
    grid=(_NBLK,),
    in_specs=[
        pl.BlockSpec((1, 1, _BLK), lambda i: (i, 0, 0)),
        pl.BlockSpec((_C, _BLK), lambda i: (0, i)),
    ],
    out_specs=pl.BlockSpec(memory_space=pltpu.SMEM),
    out_shape=jax.ShapeDtypeStruct((1, 1), jnp.float32),
    scratch_shapes=[pltpu.SMEM((1,), jnp.float32)],
)


def kernel(predict, label):
    lab3 = label.astype(jnp.int32).reshape(_NBLK, 1, _BLK)
    total = _tc_call(lab3, predict.T)
    return total[0, 0] / predict.shape[0]


# speedup vs baseline: 2.1284x; 1.0010x over previous
import functools

import jax
import jax.numpy as jnp
from jax import lax
from jax.experimental import pallas as pl
from jax.experimental.pallas import tpu as pltpu
from jax.experimental.pallas import tpu_sc as plsc

_B = 16384
_C = 1000
_BLK = 4096
_NBLK = _B // _BLK


def _tc_body(lab_ref, pred_ref, out_ref, acc_ref):
    i = pl.program_id(0)
    lab = lab_ref[0, 0, :]
    rows = lax.broadcasted_iota(jnp.int32, (_C, _BLK), 0)
    sel = rows == lab[None, :]
    part = jnp.sum(jnp.where(sel, pred_ref[...], 0.0))

    @pl.when(i == 0)
    def _():
        acc_ref[0] = 0.0

    acc_ref[0] += part

    @pl.when(i == _NBLK - 1)
    def _():
        out_ref[0, 0] = acc_ref[0]


_tc_call = pl.pallas_call(
    _tc_body,
    grid=(_NBLK,),
    in_specs=[
        pl.BlockSpec((1, 1, _BLK), lambda i: (i, 0, 0)),
        pl.BlockSpec((_C, _BLK), lambda i: (0, i)),
    ],
    out_specs=pl.BlockSpec(memory_space=pltpu.SMEM),
    out_shape=jax.ShapeDtypeStruct((1, 1), jnp.float32),
    scratch_shapes=[pltpu.SMEM((1,), jnp.float32)],
)


def kernel(predict, label):
    lab3 = label.astype(jnp.int32).reshape(_NBLK, 1, _BLK)
    total = _tc_call(lab3, predict.T)
    return total[0, 0] / predict.shape[0]
